# baseline (speedup 1.0000x reference)
import functools

import jax
import jax.numpy as jnp
from jax import lax
from jax.experimental import pallas as pl
from jax.experimental.pallas import tpu as pltpu
from jax.experimental.pallas import tpu_sc as plsc

N = 10000
E = 320000
D = 128
NC = 2
NS = 16
NW = NC * NS
EPW = E // NW
CH = 80
NPASS = 5
CPP = EPW // (CH * NPASS)
ROWS_MAIN = 632
ROWS_LAST = N - (NS - 1) * ROWS_MAIN

_MESH = plsc.VectorSubcoreMesh(core_axis_name="c", subcore_axis_name="s")


@functools.partial(
    pl.kernel,
    mesh=_MESH,
    out_type=jax.ShapeDtypeStruct((NC, N), jnp.float32),
    scratch_types=[
        pltpu.VMEM((CPP, CH), jnp.int32),
        pltpu.VMEM((CH,), jnp.float32),
        pltpu.VMEM_SHARED((N,), jnp.float32),
        pltpu.SemaphoreType.DMA,
    ],
)
def _degree_kernel(src_hbm, zeros_hbm, out_hbm, idx_v, ones_v, deg_sh, sem):
    c = lax.axis_index("c")
    s = lax.axis_index("s")
    w = s * NC + c

    for j in range(CH // 16):
        ones_v[pl.ds(16 * j, 16)] = jnp.ones((16,), jnp.float32)
    ones_v[pl.ds(CH - 16, 16)] = jnp.ones((16,), jnp.float32)

    @pl.when(s == 0)
    def _():
        pltpu.sync_copy(zeros_hbm, deg_sh)

    plsc.subcore_barrier()

    def body(i, carry):
        pltpu.async_copy(ones_v, deg_sh.at[idx_v.at[i]], sem, add=True)
        return carry

    def one_pass(p, carry):
        pltpu.sync_copy(src_hbm.at[w, p], idx_v)
        lax.fori_loop(0, CPP, body, 0)

        def drain(i, c2):
            pltpu.make_async_copy(ones_v, deg_sh.at[idx_v.at[i]], sem).wait()
            return c2

        lax.fori_loop(0, CPP, drain, 0)
        return carry

    lax.fori_loop(0, NPASS, one_pass, 0)
    plsc.subcore_barrier()

    @pl.when(s == 0)
    def _():
        pltpu.sync_copy(deg_sh, out_hbm.at[c])


def _prescale_body(x_ref, d0_ref, d1_ref, z_ref):
    g = lax.rsqrt(d0_ref[...] + d1_ref[...])
    z_ref[...] = x_ref[...] * g


_prescale = pl.pallas_call(
    _prescale_body,
    out_shape=jax.ShapeDtypeStruct((N, D), jnp.float32),
)


@functools.partial(
    pl.kernel,
    mesh=_MESH,
    out_type=jax.ShapeDtypeStruct((NC, N, D), jnp.float32),
    scratch_types=[
        pltpu.VMEM((CPP, CH), jnp.int32),
        pltpu.VMEM((CPP, CH), jnp.int32),
        pltpu.VMEM((CH, D), jnp.float32),
        pltpu.VMEM((CH, D), jnp.float32),
        pltpu.VMEM((CH, D), jnp.float32),
        pltpu.VMEM((CH, D), jnp.float32),
        pltpu.VMEM_SHARED((N, D), jnp.float32),
        pltpu.SemaphoreType.DMA,
        pltpu.SemaphoreType.DMA,
        pltpu.SemaphoreType.DMA,
        pltpu.SemaphoreType.DMA,
        pltpu.SemaphoreType.DMA,
        pltpu.SemaphoreType.DMA,
        pltpu.SemaphoreType.DMA,
        pltpu.SemaphoreType.DMA,
    ],
)
def _aggregate_kernel(z_hbm, src_hbm, dst_hbm, zeros_hbm, out_hbm,
                      src_v, dst_v, rows0, rows1, rows2, rows3, acc_sh,
                      g0, g1, g2, g3, s0, s1, s2, s3):
    c = lax.axis_index("c")
    s = lax.axis_index("s")
    w = s * NC + c

    @pl.when(s < NS - 1)
    def _():
        pltpu.sync_copy(zeros_hbm.at[pl.ds(s * ROWS_MAIN, ROWS_MAIN)],
                        acc_sh.at[pl.ds(s * ROWS_MAIN, ROWS_MAIN)])

    @pl.when(s == NS - 1)
    def _():
        pltpu.sync_copy(zeros_hbm.at[pl.ds((NS - 1) * ROWS_MAIN, ROWS_LAST)],
                        acc_sh.at[pl.ds((NS - 1) * ROWS_MAIN, ROWS_LAST)])

    plsc.subcore_barrier()

    def gather_start(i, rows, sem):
        pltpu.async_copy(z_hbm.at[src_v.at[i]], rows, sem)

    def gather_wait(i, rows, sem):
        pltpu.make_async_copy(z_hbm.at[src_v.at[i]], rows, sem).wait()

    def scatter_start(i, rows, sem):
        pltpu.async_copy(rows, acc_sh.at[dst_v.at[i]], sem, add=True)

    def scatter_wait(i, rows, sem):
        pltpu.make_async_copy(rows, acc_sh.at[dst_v.at[i]], sem).wait()

    def body(j, carry):
        i0 = 4 * j
        gather_wait(i0, rows0, g0)

        @pl.when(j > 0)
        def _():
            scatter_wait(i0 - 2, rows2, s2)

        gather_start(i0 + 2, rows2, g2)
        scatter_start(i0, rows0, s0)
        gather_wait(i0 + 1, rows1, g1)

        @pl.when(j > 0)
        def _():
            scatter_wait(i0 - 1, rows3, s3)

        gather_start(i0 + 3, rows3, g3)
        scatter_start(i0 + 1, rows1, s1)
        gather_wait(i0 + 2, rows2, g2)
        scatter_wait(i0, rows0, s0)
        gather_start(i0 + 4, rows0, g0)
        scatter_start(i0 + 2, rows2, s2)
        gather_wait(i0 + 3, rows3, g3)
        scatter_wait(i0 + 1, rows1, s1)

        @pl.when(j < CPP // 4 - 1)
        def _():
            gather_start(i0 + 5, rows1, g1)

        scatter_start(i0 + 3, rows3, s3)
        return carry

    def one_pass(p, carry):
        pltpu.sync_copy(src_hbm.at[w, p], src_v)
        pltpu.sync_copy(dst_hbm.at[w, p], dst_v)
        gather_start(0, rows0, g0)
        gather_start(1, rows1, g1)
        lax.fori_loop(0, CPP // 4, body, 0)
        last = CPP - 1
        gather_wait(last, rows0, g0)
        scatter_wait(last - 2, rows2, s2)
        scatter_start(last, rows0, s0)
        scatter_wait(last - 1, rows3, s3)
        scatter_wait(last, rows0, s0)
        return carry

    lax.fori_loop(0, NPASS, one_pass, 0)
    plsc.subcore_barrier()

    @pl.when(s < NS - 1)
    def _():
        pltpu.sync_copy(acc_sh.at[pl.ds(s * ROWS_MAIN, ROWS_MAIN)],
                        out_hbm.at[c, pl.ds(s * ROWS_MAIN, ROWS_MAIN)])

    @pl.when(s == NS - 1)
    def _():
        pltpu.sync_copy(acc_sh.at[pl.ds((NS - 1) * ROWS_MAIN, ROWS_LAST)],
                        out_hbm.at[c, pl.ds((NS - 1) * ROWS_MAIN, ROWS_LAST)])


def _finish_body(parts_ref, d0_ref, d1_ref, out_ref):
    g = lax.rsqrt(d0_ref[...] + d1_ref[...])
    out_ref[...] = g * (parts_ref[0] + parts_ref[1])


_finish = pl.pallas_call(
    _finish_body,
    out_shape=jax.ShapeDtypeStruct((N, D), jnp.float32),
)


def kernel(entity_embed, edge_index):
    src = edge_index[0].astype(jnp.int32).reshape(NW, NPASS, CPP, CH)
    dst = edge_index[1].astype(jnp.int32).reshape(NW, NPASS, CPP, CH)

    deg_parts = _degree_kernel(src, jnp.zeros((N,), jnp.float32))
    d0 = deg_parts[0].reshape(N, 1)
    d1 = deg_parts[1].reshape(N, 1)

    z = _prescale(entity_embed, d0, d1)

    parts = _aggregate_kernel(z, src, dst, jnp.zeros((N, D), jnp.float32))
    return _finish(parts, d0, d1)


# speedup vs baseline: 1.1261x; 1.1261x over previous
import functools

import jax
import jax.numpy as jnp
from jax import lax
from jax.experimental import pallas as pl
from jax.experimental.pallas import tpu as pltpu
from jax.experimental.pallas import tpu_sc as plsc

N = 10000
E = 320000
D = 128
NC = 2
NS = 16
NW = NC * NS
EPW = E // NW
CH = 100
NPASS = 4
CPP = EPW // (CH * NPASS)
ROWS_MAIN = 632
ROWS_LAST = N - (NS - 1) * ROWS_MAIN

_MESH = plsc.VectorSubcoreMesh(core_axis_name="c", subcore_axis_name="s")


@functools.partial(
    pl.kernel,
    mesh=_MESH,
    out_type=jax.ShapeDtypeStruct((NC, N), jnp.float32),
    scratch_types=[
        pltpu.VMEM((CPP, CH), jnp.int32),
        pltpu.VMEM((CH,), jnp.float32),
        pltpu.VMEM_SHARED((N,), jnp.float32),
        pltpu.SemaphoreType.DMA,
    ],
)
def _degree_kernel(src_hbm, zeros_hbm, out_hbm, idx_v, ones_v, deg_sh, sem):
    c = lax.axis_index("c")
    s = lax.axis_index("s")
    w = s * NC + c

    for j in range(CH // 16):
        ones_v[pl.ds(16 * j, 16)] = jnp.ones((16,), jnp.float32)
    ones_v[pl.ds(CH - 16, 16)] = jnp.ones((16,), jnp.float32)

    @pl.when(s == 0)
    def _():
        pltpu.sync_copy(zeros_hbm, deg_sh)

    plsc.subcore_barrier()

    def body(i, carry):
        pltpu.async_copy(ones_v, deg_sh.at[idx_v.at[i]], sem, add=True)
        return carry

    def one_pass(p, carry):
        pltpu.sync_copy(src_hbm.at[w, p], idx_v)
        lax.fori_loop(0, CPP, body, 0)

        def drain(i, c2):
            pltpu.make_async_copy(ones_v, deg_sh.at[idx_v.at[i]], sem).wait()
            return c2

        lax.fori_loop(0, CPP, drain, 0)
        return carry

    lax.fori_loop(0, NPASS, one_pass, 0)
    plsc.subcore_barrier()

    @pl.when(s == 0)
    def _():
        pltpu.sync_copy(deg_sh, out_hbm.at[c])


def _prescale_body(x_ref, d0_ref, d1_ref, z_ref):
    g = lax.rsqrt(d0_ref[...] + d1_ref[...])
    z_ref[...] = x_ref[...] * g


_prescale = pl.pallas_call(
    _prescale_body,
    out_shape=jax.ShapeDtypeStruct((N, D), jnp.float32),
)


@functools.partial(
    pl.kernel,
    mesh=_MESH,
    out_type=jax.ShapeDtypeStruct((NC, N, D), jnp.float32),
    scratch_types=[
        pltpu.VMEM((CPP, CH), jnp.int32),
        pltpu.VMEM((CPP, CH), jnp.int32),
        pltpu.VMEM((CH, D), jnp.float32),
        pltpu.VMEM((CH, D), jnp.float32),
        pltpu.VMEM((CH, D), jnp.float32),
        pltpu.VMEM_SHARED((N, D), jnp.float32),
        pltpu.SemaphoreType.DMA,
        pltpu.SemaphoreType.DMA,
        pltpu.SemaphoreType.DMA,
        pltpu.SemaphoreType.DMA,
        pltpu.SemaphoreType.DMA,
        pltpu.SemaphoreType.DMA,
    ],
)
def _aggregate_kernel(z_hbm, src_hbm, dst_hbm, zeros_hbm, out_hbm,
                      src_v, dst_v, rows0, rows1, rows2, acc_sh,
                      g0, g1, g2, s0, s1, s2):
    c = lax.axis_index("c")
    s = lax.axis_index("s")
    w = s * NC + c

    @pl.when(s < NS - 1)
    def _():
        pltpu.sync_copy(zeros_hbm.at[pl.ds(s * ROWS_MAIN, ROWS_MAIN)],
                        acc_sh.at[pl.ds(s * ROWS_MAIN, ROWS_MAIN)])

    @pl.when(s == NS - 1)
    def _():
        pltpu.sync_copy(zeros_hbm.at[pl.ds((NS - 1) * ROWS_MAIN, ROWS_LAST)],
                        acc_sh.at[pl.ds((NS - 1) * ROWS_MAIN, ROWS_LAST)])

    plsc.subcore_barrier()

    def gather_start(i, rows, sem):
        pltpu.async_copy(z_hbm.at[src_v.at[i]], rows, sem)

    def gather_wait(i, rows, sem):
        pltpu.make_async_copy(z_hbm.at[src_v.at[i]], rows, sem).wait()

    def scatter_start(i, rows, sem):
        pltpu.async_copy(rows, acc_sh.at[dst_v.at[i]], sem, add=True)

    def scatter_wait(i, rows, sem):
        pltpu.make_async_copy(rows, acc_sh.at[dst_v.at[i]], sem).wait()

    def body(j, carry):
        i0 = 3 * j
        gather_wait(i0, rows0, g0)

        @pl.when(j > 0)
        def _():
            scatter_wait(i0 - 1, rows2, s2)

        gather_start(i0 + 2, rows2, g2)
        scatter_start(i0, rows0, s0)
        gather_wait(i0 + 1, rows1, g1)
        scatter_wait(i0, rows0, s0)
        gather_start(i0 + 3, rows0, g0)
        scatter_start(i0 + 1, rows1, s1)
        gather_wait(i0 + 2, rows2, g2)
        scatter_wait(i0 + 1, rows1, s1)

        @pl.when(j < CPP // 3 - 1)
        def _():
            gather_start(i0 + 4, rows1, g1)

        scatter_start(i0 + 2, rows2, s2)
        return carry

    def one_pass(p, carry):
        pltpu.sync_copy(src_hbm.at[w, p], src_v)
        pltpu.sync_copy(dst_hbm.at[w, p], dst_v)
        gather_start(0, rows0, g0)
        gather_start(1, rows1, g1)
        lax.fori_loop(0, CPP // 3, body, 0)
        last = CPP - 1
        gather_wait(last, rows0, g0)
        scatter_wait(last - 1, rows2, s2)
        scatter_start(last, rows0, s0)
        scatter_wait(last, rows0, s0)
        return carry

    lax.fori_loop(0, NPASS, one_pass, 0)
    plsc.subcore_barrier()

    @pl.when(s < NS - 1)
    def _():
        pltpu.sync_copy(acc_sh.at[pl.ds(s * ROWS_MAIN, ROWS_MAIN)],
                        out_hbm.at[c, pl.ds(s * ROWS_MAIN, ROWS_MAIN)])

    @pl.when(s == NS - 1)
    def _():
        pltpu.sync_copy(acc_sh.at[pl.ds((NS - 1) * ROWS_MAIN, ROWS_LAST)],
                        out_hbm.at[c, pl.ds((NS - 1) * ROWS_MAIN, ROWS_LAST)])


def _finish_body(parts_ref, d0_ref, d1_ref, out_ref):
    g = lax.rsqrt(d0_ref[...] + d1_ref[...])
    out_ref[...] = g * (parts_ref[0] + parts_ref[1])


_finish = pl.pallas_call(
    _finish_body,
    out_shape=jax.ShapeDtypeStruct((N, D), jnp.float32),
)


def kernel(entity_embed, edge_index):
    src = edge_index[0].astype(jnp.int32).reshape(NW, NPASS, CPP, CH)
    dst = edge_index[1].astype(jnp.int32).reshape(NW, NPASS, CPP, CH)

    deg_parts = _degree_kernel(src, jnp.zeros((N,), jnp.float32))
    d0 = deg_parts[0].reshape(N, 1)
    d1 = deg_parts[1].reshape(N, 1)

    z = _prescale(entity_embed, d0, d1)

    parts = _aggregate_kernel(z, src, dst, jnp.zeros((N, D), jnp.float32))
    return _finish(parts, d0, d1)
